# baseline (speedup 1.0000x reference)
import functools

import jax
import jax.numpy as jnp
from jax import lax
from jax.experimental import pallas as pl
from jax.experimental.pallas import tpu as pltpu
from jax.experimental.pallas import tpu_sc as plsc

N = 10000
E = 320000
D_IN = 128
H = 64
LAT = 32

NC = 2
NS = 16
DPT = H // NS
EC = 2048
NCHB = 80
NCHA = NCHB + 1
EPS = NCHB * EC
EP = NC * EPS

_mesh = plsc.VectorSubcoreMesh(
    core_axis_name="c", subcore_axis_name="s", num_cores=NC, num_subcores=NS)


def _edge_compute(yT_l, accT_l, sbuf, dbuf, wbuf):
    @plsc.parallel_loop(0, EC // 16, unroll=2)
    def _(g):
        s16 = sbuf[pl.ds(g * 16, 16)]
        d16 = dbuf[pl.ds(g * 16, 16)]
        w16 = wbuf[pl.ds(g * 16, 16)]
        for q in range(DPT):
            qidx = jnp.full((16,), q, dtype=jnp.int32)
            v = plsc.load_gather(yT_l, [qidx, s16])
            plsc.addupdate_scatter(accT_l, [qidx, d16], v * w16)


def _sc_edge_agg_body(yT_hbm, src_hbm, dst_hbm, ew_hbm, out_hbm,
                      yT_l, accT_l, sA, dA, wA, sB, dB, wB, semA, semB):
    c = lax.axis_index("c")
    s = lax.axis_index("s")
    pltpu.sync_copy(yT_hbm.at[pl.ds(s * DPT, DPT)], yT_l)
    for q in range(DPT):
        def zbody(i, carry, q=q):
            accT_l[q, pl.ds(i * 16, 16)] = jnp.zeros((16,), jnp.float32)
            return carry
        lax.fori_loop(0, N // 16, zbody, 0)

    def fire(j, sb, db, wb, sem):
        pltpu.async_copy(src_hbm.at[c, j], sb, sem)
        pltpu.async_copy(dst_hbm.at[c, j], db, sem)
        pltpu.async_copy(ew_hbm.at[c, j], wb, sem)

    def drain(j, sb, db, wb, sem):
        pltpu.make_async_copy(src_hbm.at[c, j], sb, sem).wait()
        pltpu.make_async_copy(dst_hbm.at[c, j], db, sem).wait()
        pltpu.make_async_copy(ew_hbm.at[c, j], wb, sem).wait()

    fire(0, sA, dA, wA, semA)

    def pipe_body(jj, carry):
        c0 = jj * 2
        drain(c0, sA, dA, wA, semA)
        fire(c0 + 1, sB, dB, wB, semB)
        _edge_compute(yT_l, accT_l, sA, dA, wA)
        drain(c0 + 1, sB, dB, wB, semB)
        fire(c0 + 2, sA, dA, wA, semA)
        _edge_compute(yT_l, accT_l, sB, dB, wB)
        return carry

    lax.fori_loop(0, NCHB // 2, pipe_body, 0)
    drain(NCHB, sA, dA, wA, semA)
    pltpu.sync_copy(accT_l, out_hbm.at[c, pl.ds(s * DPT, DPT)])


_sc_edge_agg = functools.partial(
    pl.kernel,
    mesh=_mesh,
    out_type=jax.ShapeDtypeStruct((NC, H, N), jnp.float32),
    scratch_types=[
        pltpu.VMEM((DPT, N), jnp.float32),
        pltpu.VMEM((DPT, N), jnp.float32),
        pltpu.VMEM((EC,), jnp.int32),
        pltpu.VMEM((EC,), jnp.int32),
        pltpu.VMEM((EC,), jnp.float32),
        pltpu.VMEM((EC,), jnp.int32),
        pltpu.VMEM((EC,), jnp.int32),
        pltpu.VMEM((EC,), jnp.float32),
        pltpu.SemaphoreType.DMA,
        pltpu.SemaphoreType.DMA,
    ],
    compiler_params=pltpu.CompilerParams(
        use_tc_tiling_on_sc=False, needs_layout_passes=False),
)(_sc_edge_agg_body)


def _proj_t_body(x_ref, wrel_ref, wroot_ref, yt_ref, rt_ref):
    x = x_ref[...]
    yt_ref[...] = lax.dot_general(wrel_ref[...], x, (((1,), (1,)), ((), ())),
                                  preferred_element_type=jnp.float32)
    rt_ref[...] = lax.dot_general(wroot_ref[...], x, (((1,), (1,)), ((), ())),
                                  preferred_element_type=jnp.float32)


def _proj_t(x, w_rel, w_root, d_in):
    return pl.pallas_call(
        _proj_t_body,
        in_specs=[
            pl.BlockSpec((N, d_in), lambda: (0, 0)),
            pl.BlockSpec((H, d_in), lambda: (0, 0)),
            pl.BlockSpec((H, d_in), lambda: (0, 0)),
        ],
        out_specs=[
            pl.BlockSpec((H, N), lambda: (0, 0)),
            pl.BlockSpec((H, N), lambda: (0, 0)),
        ],
        out_shape=[
            jax.ShapeDtypeStruct((H, N), jnp.float32),
            jax.ShapeDtypeStruct((H, N), jnp.float32),
        ],
    )(x, w_rel, w_root)


def _fuse_t_body(agg_ref, rt_ref, b_ref, wrel_ref, wroot_ref,
                 y2t_ref, r2t_ref):
    h1t = jnp.maximum(agg_ref[0] + agg_ref[1] + rt_ref[...] + b_ref[...], 0.0)
    y2t_ref[...] = lax.dot_general(wrel_ref[...], h1t, (((1,), (0,)), ((), ())),
                                   preferred_element_type=jnp.float32)
    r2t_ref[...] = lax.dot_general(wroot_ref[...], h1t, (((1,), (0,)), ((), ())),
                                   preferred_element_type=jnp.float32)


def _fuse_t(aggT, r1T, b1, w_rel, w_root):
    return pl.pallas_call(
        _fuse_t_body,
        in_specs=[
            pl.BlockSpec((NC, H, N), lambda: (0, 0, 0)),
            pl.BlockSpec((H, N), lambda: (0, 0)),
            pl.BlockSpec((H, 1), lambda: (0, 0)),
            pl.BlockSpec((H, H), lambda: (0, 0)),
            pl.BlockSpec((H, H), lambda: (0, 0)),
        ],
        out_specs=[
            pl.BlockSpec((H, N), lambda: (0, 0)),
            pl.BlockSpec((H, N), lambda: (0, 0)),
        ],
        out_shape=[
            jax.ShapeDtypeStruct((H, N), jnp.float32),
            jax.ShapeDtypeStruct((H, N), jnp.float32),
        ],
    )(aggT, r1T, b1, w_rel, w_root)


def _relu_t_body(agg_ref, rt_ref, b_ref, ht_ref):
    ht_ref[...] = jnp.maximum(
        agg_ref[0] + agg_ref[1] + rt_ref[...] + b_ref[...], 0.0)


def _relu_t(aggT, r2T, b2):
    return pl.pallas_call(
        _relu_t_body,
        in_specs=[
            pl.BlockSpec((NC, H, N), lambda: (0, 0, 0)),
            pl.BlockSpec((H, N), lambda: (0, 0)),
            pl.BlockSpec((H, 1), lambda: (0, 0)),
        ],
        out_specs=pl.BlockSpec((H, N), lambda: (0, 0)),
        out_shape=jax.ShapeDtypeStruct((H, N), jnp.float32),
    )(aggT, r2T, b2)


FB = 64000
NFB = (N * H) // FB


def _heads_body(flat_ref, wmu_ref, wlv_ref, bmu_ref, blv_ref,
                mu_ref, lv_ref):
    @pl.when(pl.program_id(0) == 0)
    def _():
        mu_ref[...] = bmu_ref[...]
        lv_ref[...] = blv_ref[...]
    f = flat_ref[...]
    mu_ref[...] += lax.dot_general(f, wmu_ref[...], (((1,), (1,)), ((), ())),
                                   preferred_element_type=jnp.float32)
    lv_ref[...] += lax.dot_general(f, wlv_ref[...], (((1,), (1,)), ((), ())),
                                   preferred_element_type=jnp.float32)


def _heads(flat, w_mu, w_lv, b_mu, b_lv):
    return pl.pallas_call(
        _heads_body,
        grid=(NFB,),
        in_specs=[
            pl.BlockSpec((1, FB), lambda i: (0, i)),
            pl.BlockSpec((LAT, FB), lambda i: (0, i)),
            pl.BlockSpec((LAT, FB), lambda i: (0, i)),
            pl.BlockSpec((1, LAT), lambda i: (0, 0)),
            pl.BlockSpec((1, LAT), lambda i: (0, 0)),
        ],
        out_specs=[
            pl.BlockSpec((1, LAT), lambda i: (0, 0)),
            pl.BlockSpec((1, LAT), lambda i: (0, 0)),
        ],
        out_shape=[
            jax.ShapeDtypeStruct((1, LAT), jnp.float32),
            jax.ShapeDtypeStruct((1, LAT), jnp.float32),
        ],
    )(flat, w_mu, w_lv, b_mu, b_lv)


def kernel(x, edge_index, edge_weight, batch,
           W1_rel, b1_rel, W1_root, W2_rel, b2_rel, W2_root,
           W_mu, b_mu, W_lv, b_lv):
    src = edge_index[0].astype(jnp.int32)
    dst = edge_index[1].astype(jnp.int32)
    ew = edge_weight.astype(jnp.float32)
    pad = EP - E
    pad_idx = (jnp.arange(pad, dtype=jnp.int32) * 37) % N
    dummy = jnp.broadcast_to((jnp.arange(EC, dtype=jnp.int32) * 41) % N,
                             (NC, 1, EC))
    src3 = jnp.concatenate(
        [jnp.concatenate([src, pad_idx]).reshape(NC, NCHB, EC), dummy], 1)
    dst3 = jnp.concatenate(
        [jnp.concatenate([dst, pad_idx]).reshape(NC, NCHB, EC), dummy], 1)
    ew3 = jnp.concatenate(
        [jnp.concatenate([ew, jnp.zeros((pad,), jnp.float32)]
                         ).reshape(NC, NCHB, EC),
         jnp.zeros((NC, 1, EC), jnp.float32)], 1)
    b1 = b1_rel.reshape(H, 1)
    b2 = b2_rel.reshape(H, 1)

    y1T, r1T = _proj_t(x, W1_rel, W1_root, D_IN)
    agg1T = _sc_edge_agg(y1T, src3, dst3, ew3)
    y2T, r2T = _fuse_t(agg1T, r1T, b1, W2_rel, W2_root)

    agg2T = _sc_edge_agg(y2T, src3, dst3, ew3)
    h2T = _relu_t(agg2T, r2T, b2)

    flat = h2T.T.reshape(1, N * H)
    mu2, lv2 = _heads(flat, W_mu, W_lv,
                      b_mu.reshape(1, LAT), b_lv.reshape(1, LAT))
    return mu2.reshape(LAT), lv2.reshape(LAT)


# speedup vs baseline: 1.1977x; 1.1977x over previous
import functools

import jax
import jax.numpy as jnp
from jax import lax
from jax.experimental import pallas as pl
from jax.experimental.pallas import tpu as pltpu
from jax.experimental.pallas import tpu_sc as plsc

N = 10000
E = 320000
D_IN = 128
H = 64
LAT = 32

NC = 2
NS = 16
NW = NC * NS
CHUNK = 128
EDGES_PER_TILE = -(-E // NW)
NCH = 80
NCHA = NCH + 1
EP = NW * NCH * CHUNK
EPA = NW * NCHA * CHUNK
ROWS_PER_TILE = 632
NP_ROWS = NS * ROWS_PER_TILE

_mesh = plsc.VectorSubcoreMesh(
    core_axis_name="c", subcore_axis_name="s", num_cores=NC, num_subcores=NS)


_PERM = tuple(base + off
              for base in (0, 32)
              for i in range(16)
              for off in (i, 16 + i))


def _scale_rows(rows_bf, rows_f, ew_v, j):
    @plsc.parallel_loop(0, CHUNK // 16, unroll=2)
    def _(g):
        ew_vec = ew_v[j, pl.ds(g * 16, 16)]
        for l in range(16):
            e = g * 16 + l
            idx = jnp.full((16,), l, dtype=jnp.int32)
            wsplat = ew_vec.at[idx].get(mode="promise_in_bounds")
            for half in range(2):
                ab = rows_bf[e, pl.ds(half * 32, 32)]
                a, b = plsc.unpack(ab, format=plsc.PackFormat.INTERLEAVED)
                rows_f[e, pl.ds(half * 32, 16)] = (
                    a.astype(jnp.float32) * wsplat)
                rows_f[e, pl.ds(half * 32 + 16, 16)] = (
                    b.astype(jnp.float32) * wsplat)


def _sc_edge_agg_body(y_hbm, src_hbm, dst_hbm, ew_hbm, z_hbm, out_hbm,
                      src_v, dst_v, ew_v, rbf0, rbf1, rows0, rows1, ysp, acc,
                      gsem0, gsem1, ssem0, ssem1):
    c = lax.axis_index("c")
    s = lax.axis_index("s")
    w = s * NC + c
    pltpu.sync_copy(z_hbm.at[pl.ds(s * ROWS_PER_TILE, ROWS_PER_TILE)],
                    acc.at[pl.ds(s * ROWS_PER_TILE, ROWS_PER_TILE)])
    pltpu.sync_copy(y_hbm.at[pl.ds(s * (N // NS), N // NS)],
                    ysp.at[pl.ds(s * (N // NS), N // NS)])
    pltpu.sync_copy(src_hbm.at[w], src_v)
    pltpu.sync_copy(dst_hbm.at[w], dst_v)
    pltpu.sync_copy(ew_hbm.at[w], ew_v)
    plsc.subcore_barrier()

    def gather_wait(rbf, gsem, j):
        pltpu.make_async_copy(ysp.at[src_v.at[j]], rbf, gsem).wait()

    def scatter_wait(rows, ssem, j):
        pltpu.make_async_copy(rows, acc.at[dst_v.at[j]], ssem).wait()

    pltpu.async_copy(ysp.at[src_v.at[0]], rbf0, gsem0)

    def pipe_body(jj, carry):
        c0 = jj * 2
        c1 = c0 + 1
        c2 = c0 + 2
        gather_wait(rbf0, gsem0, c0)
        _scale_rows(rbf0, rows0, ew_v, c0)

        @pl.when(jj > 0)
        def _():
            scatter_wait(rows1, ssem1, c1 - 2)

        pltpu.async_copy(ysp.at[src_v.at[c1]], rbf1, gsem1)
        pltpu.async_copy(rows0, acc.at[dst_v.at[c0]], ssem0, add=True)
        gather_wait(rbf1, gsem1, c1)
        _scale_rows(rbf1, rows1, ew_v, c1)
        scatter_wait(rows0, ssem0, c0)
        pltpu.async_copy(ysp.at[src_v.at[c2]], rbf0, gsem0)
        pltpu.async_copy(rows1, acc.at[dst_v.at[c1]], ssem1, add=True)
        return carry

    lax.fori_loop(0, NCH // 2, pipe_body, 0)
    gather_wait(rbf0, gsem0, NCH)
    scatter_wait(rows1, ssem1, NCH - 1)
    plsc.subcore_barrier()
    pltpu.sync_copy(acc.at[pl.ds(s * ROWS_PER_TILE, ROWS_PER_TILE)],
                    out_hbm.at[c, pl.ds(s * ROWS_PER_TILE, ROWS_PER_TILE)])


_sc_edge_agg = functools.partial(
    pl.kernel,
    mesh=_mesh,
    out_type=jax.ShapeDtypeStruct((NC, NP_ROWS, H), jnp.float32),
    scratch_types=[
        pltpu.VMEM((NCHA, CHUNK), jnp.int32),
        pltpu.VMEM((NCHA, CHUNK), jnp.int32),
        pltpu.VMEM((NCHA, CHUNK), jnp.float32),
        pltpu.VMEM((CHUNK, H), jnp.bfloat16),
        pltpu.VMEM((CHUNK, H), jnp.bfloat16),
        pltpu.VMEM((CHUNK, H), jnp.float32),
        pltpu.VMEM((CHUNK, H), jnp.float32),
        pltpu.VMEM_SHARED((N, H), jnp.bfloat16),
        pltpu.VMEM_SHARED((NP_ROWS, H), jnp.float32),
        pltpu.SemaphoreType.DMA,
        pltpu.SemaphoreType.DMA,
        pltpu.SemaphoreType.DMA,
        pltpu.SemaphoreType.DMA,
    ],
    compiler_params=pltpu.CompilerParams(
        use_tc_tiling_on_sc=False, needs_layout_passes=False),
)(_sc_edge_agg_body)


NB = 10
BN = N // NB


def _proj_body(x_ref, wrel_ref, wroot_ref, y_ref, r_ref):
    x = x_ref[...]
    y_ref[...] = lax.dot_general(x, wrel_ref[...], (((1,), (1,)), ((), ())),
                                 preferred_element_type=jnp.float32
                                 ).astype(jnp.bfloat16)
    r_ref[...] = lax.dot_general(x, wroot_ref[...], (((1,), (1,)), ((), ())),
                                 preferred_element_type=jnp.float32)


def _proj(x, w_rel, w_root, d_in):
    return pl.pallas_call(
        _proj_body,
        grid=(NB,),
        in_specs=[
            pl.BlockSpec((BN, d_in), lambda i: (i, 0)),
            pl.BlockSpec((H, d_in), lambda i: (0, 0)),
            pl.BlockSpec((H, d_in), lambda i: (0, 0)),
        ],
        out_specs=[
            pl.BlockSpec((BN, H), lambda i: (i, 0)),
            pl.BlockSpec((BN, H), lambda i: (i, 0)),
        ],
        out_shape=[
            jax.ShapeDtypeStruct((N, H), jnp.bfloat16),
            jax.ShapeDtypeStruct((N, H), jnp.float32),
        ],
    )(x, w_rel, w_root)


def _fuse_proj_body(agg_ref, r_ref, b_ref, wrel_ref, wroot_ref,
                    y2_ref, r2_ref):
    h1 = jnp.maximum(agg_ref[0] + agg_ref[1] + r_ref[...] + b_ref[...], 0.0)
    y2_ref[...] = lax.dot_general(h1, wrel_ref[...], (((1,), (1,)), ((), ())),
                                  preferred_element_type=jnp.float32
                                  ).astype(jnp.bfloat16)
    r2_ref[...] = lax.dot_general(h1, wroot_ref[...], (((1,), (1,)), ((), ())),
                                  preferred_element_type=jnp.float32)


def _fuse_proj(aggpair, r1, b1, w_rel, w_root):
    return pl.pallas_call(
        _fuse_proj_body,
        grid=(NB,),
        in_specs=[
            pl.BlockSpec((NC, BN, H), lambda i: (0, i, 0)),
            pl.BlockSpec((BN, H), lambda i: (i, 0)),
            pl.BlockSpec((1, H), lambda i: (0, 0)),
            pl.BlockSpec((H, H), lambda i: (0, 0)),
            pl.BlockSpec((H, H), lambda i: (0, 0)),
        ],
        out_specs=[
            pl.BlockSpec((BN, H), lambda i: (i, 0)),
            pl.BlockSpec((BN, H), lambda i: (i, 0)),
        ],
        out_shape=[
            jax.ShapeDtypeStruct((N, H), jnp.bfloat16),
            jax.ShapeDtypeStruct((N, H), jnp.float32),
        ],
    )(aggpair, r1, b1, w_rel, w_root)


def _relu_fuse_body(agg_ref, r_ref, b_ref, h_ref):
    h_ref[...] = jnp.maximum(
        agg_ref[0] + agg_ref[1] + r_ref[...] + b_ref[...], 0.0)


def _relu_fuse(aggpair, r2, b2):
    return pl.pallas_call(
        _relu_fuse_body,
        grid=(NB,),
        in_specs=[
            pl.BlockSpec((NC, BN, H), lambda i: (0, i, 0)),
            pl.BlockSpec((BN, H), lambda i: (i, 0)),
            pl.BlockSpec((1, H), lambda i: (0, 0)),
        ],
        out_specs=pl.BlockSpec((BN, H), lambda i: (i, 0)),
        out_shape=jax.ShapeDtypeStruct((N, H), jnp.float32),
    )(aggpair, r2, b2)


FB = 64000
NFB = (N * H) // FB


def _heads_body(flat_ref, wmu_ref, wlv_ref, bmu_ref, blv_ref,
                mu_ref, lv_ref):
    @pl.when(pl.program_id(0) == 0)
    def _():
        mu_ref[...] = bmu_ref[...]
        lv_ref[...] = blv_ref[...]
    f = flat_ref[...]
    mu_ref[...] += lax.dot_general(f, wmu_ref[...], (((1,), (1,)), ((), ())),
                                   preferred_element_type=jnp.float32)
    lv_ref[...] += lax.dot_general(f, wlv_ref[...], (((1,), (1,)), ((), ())),
                                   preferred_element_type=jnp.float32)


def _heads(flat, w_mu, w_lv, b_mu, b_lv):
    return pl.pallas_call(
        _heads_body,
        grid=(NFB,),
        in_specs=[
            pl.BlockSpec((1, FB), lambda i: (0, i)),
            pl.BlockSpec((LAT, FB), lambda i: (0, i)),
            pl.BlockSpec((LAT, FB), lambda i: (0, i)),
            pl.BlockSpec((1, LAT), lambda i: (0, 0)),
            pl.BlockSpec((1, LAT), lambda i: (0, 0)),
        ],
        out_specs=[
            pl.BlockSpec((1, LAT), lambda i: (0, 0)),
            pl.BlockSpec((1, LAT), lambda i: (0, 0)),
        ],
        out_shape=[
            jax.ShapeDtypeStruct((1, LAT), jnp.float32),
            jax.ShapeDtypeStruct((1, LAT), jnp.float32),
        ],
    )(flat, w_mu, w_lv, b_mu, b_lv)


def kernel(x, edge_index, edge_weight, batch,
           W1_rel, b1_rel, W1_root, W2_rel, b2_rel, W2_root,
           W_mu, b_mu, W_lv, b_lv):
    src = edge_index[0].astype(jnp.int32)
    dst = edge_index[1].astype(jnp.int32)
    ew = edge_weight.astype(jnp.float32)
    pad = EP - E
    pad_idx = (jnp.arange(pad, dtype=jnp.int32) * 37) % N
    dummy = jnp.broadcast_to((jnp.arange(CHUNK, dtype=jnp.int32) * 41) % N,
                             (NW, 1, CHUNK))
    src3 = jnp.concatenate(
        [jnp.concatenate([src, pad_idx]).reshape(NW, NCH, CHUNK), dummy], 1)
    dst3 = jnp.concatenate(
        [jnp.concatenate([dst, pad_idx]).reshape(NW, NCH, CHUNK), dummy], 1)
    ew3 = jnp.concatenate(
        [jnp.concatenate([ew, jnp.zeros((pad,), jnp.float32)]
                         ).reshape(NW, NCH, CHUNK),
         jnp.zeros((NW, 1, CHUNK), jnp.float32)], 1)
    zeros = jnp.zeros((NP_ROWS, H), jnp.float32)
    b1 = b1_rel.reshape(1, H)
    b2 = b2_rel.reshape(1, H)
    perm = jnp.asarray(_PERM, dtype=jnp.int32)
    W1p = W1_rel[perm]
    W2p = W2_rel[perm]

    y1, r1 = _proj(x, W1p, W1_root, D_IN)
    agg1 = _sc_edge_agg(y1, src3, dst3, ew3, zeros)
    y2, r2 = _fuse_proj(agg1, r1, b1, W2p, W2_root)

    agg2 = _sc_edge_agg(y2, src3, dst3, ew3, zeros)
    h2 = _relu_fuse(agg2, r2, b2)

    flat = h2.reshape(1, N * H)
    mu2, lv2 = _heads(flat, W_mu, W_lv,
                      b_mu.reshape(1, LAT), b_lv.reshape(1, LAT))
    return mu2.reshape(LAT), lv2.reshape(LAT)
